# baseline (speedup 1.0000x reference)
import jax
import jax.numpy as jnp
from jax import lax
from jax.experimental import pallas as pl
from jax.experimental.pallas import tpu as pltpu
from jax.experimental.pallas import tpu_sc as plsc

N, E, G, H = 10000, 320000, 32, 128
NW = 32
EPW = E // NW
CH = 125
NCH = EPW // CH
NPT = N // 16

BN = 1000
BE = 2000


def _ln_block(x, g, b, eps=1e-5):
    m = jnp.mean(x, axis=-1, keepdims=True)
    v = jnp.mean((x - m) ** 2, axis=-1, keepdims=True)
    return (x - m) / jnp.sqrt(v + eps) * g + b


def _node_pre_body(nf, g2, b2, whi, whj, h_out, pi_out, pj_out):
    h = _ln_block(nf[...], g2[...], b2[...])
    h_out[...] = h
    pi_out[...] = jnp.dot(h, whi[...], preferred_element_type=jnp.float32).astype(jnp.bfloat16)
    pj_out[...] = jnp.dot(h, whj[...], preferred_element_type=jnp.float32).astype(jnp.bfloat16)


def _node_pre(nf, g2, b2, whi, whj):
    const = pl.BlockSpec((H, H), lambda i: (0, 0))
    vec = pl.BlockSpec((1, H), lambda i: (0, 0))
    blk = pl.BlockSpec((BN, H), lambda i: (i, 0))
    return pl.pallas_call(
        _node_pre_body,
        grid=(N // BN,),
        in_specs=[blk, vec, vec, const, const],
        out_specs=[blk, blk, blk],
        out_shape=[jax.ShapeDtypeStruct((N, H), jnp.float32),
                   jax.ShapeDtypeStruct((N, H), jnp.bfloat16),
                   jax.ShapeDtypeStruct((N, H), jnp.bfloat16)],
    )(nf, g2, b2, whi, whj)


def _gather_body(pi_hbm, pj_hbm, src2d, dst2d, out_hbm, idx_s, idx_d, rows):
    wid = lax.axis_index("c") * 16 + lax.axis_index("s")
    pltpu.sync_copy(src2d.at[pl.ds(wid * NCH, NCH), :], idx_s)
    pltpu.sync_copy(dst2d.at[pl.ds(wid * NCH, NCH), :], idx_d)

    def chunk(j, _):
        eoff = wid * EPW + j * CH
        pltpu.sync_copy(pi_hbm.at[idx_s.at[j]], rows)
        pltpu.sync_copy(pj_hbm.at[idx_d.at[j]], rows, add=True)
        pltpu.sync_copy(rows, out_hbm.at[pl.ds(eoff, CH), :])
        return ()

    lax.fori_loop(0, NCH, chunk, ())


def _gather_sum(pi, pj, src2d, dst2d):
    mesh = plsc.VectorSubcoreMesh(core_axis_name="c", subcore_axis_name="s")
    fn = pl.kernel(
        _gather_body,
        out_type=jax.ShapeDtypeStruct((E, H), jnp.bfloat16),
        mesh=mesh,
        scratch_types=[
            pltpu.VMEM((NCH, CH), jnp.int32),
            pltpu.VMEM((NCH, CH), jnp.int32),
            pltpu.VMEM((CH, H), jnp.bfloat16),
        ],
        compiler_params=pltpu.CompilerParams(use_tc_tiling_on_sc=False),
    )
    return fn(pi, pj, src2d, dst2d)


def _edge_body(ef, gsum, e2g, fd, lat, wef, wlat, wfd, be1, we2, be2, out):
    qb = jnp.dot(lat[...], wlat[...], preferred_element_type=jnp.float32)
    onehot = (e2g[...] == lax.broadcasted_iota(jnp.int32, (1, G), 1)).astype(jnp.float32)
    lat_term = jnp.dot(onehot, qb, preferred_element_type=jnp.float32)
    fdv = fd[...]
    wfdv = wfd[...]
    fd_term = (fdv[:, 0:1] * wfdv[0:1, :] + fdv[:, 1:2] * wfdv[1:2, :]
               + fdv[:, 2:3] * wfdv[2:3, :])
    pre = (jnp.dot(ef[...], wef[...], preferred_element_type=jnp.float32)
           + gsum[...].astype(jnp.float32) + lat_term + fd_term + be1[...])
    e1 = jax.nn.silu(pre)
    e2 = jax.nn.silu(jnp.dot(e1, we2[...], preferred_element_type=jnp.float32)
                     + be2[...])
    out[...] = e2


def _edge_mlp(edge_feats, gsum, e2g_2d, fd, lat, wef, wlat, wfd, be1, we2, be2):
    const = pl.BlockSpec((H, H), lambda i: (0, 0))
    vec = pl.BlockSpec((1, H), lambda i: (0, 0))
    eblk = pl.BlockSpec((BE, H), lambda i: (i, 0))
    return pl.pallas_call(
        _edge_body,
        grid=(E // BE,),
        in_specs=[
            eblk,
            eblk,
            pl.BlockSpec((BE, 1), lambda i: (i, 0)),
            pl.BlockSpec((BE, 3), lambda i: (i, 0)),
            pl.BlockSpec((G, 6), lambda i: (0, 0)),
            const,
            pl.BlockSpec((6, H), lambda i: (0, 0)),
            pl.BlockSpec((3, H), lambda i: (0, 0)),
            vec,
            const,
            vec,
        ],
        out_specs=eblk,
        out_shape=jax.ShapeDtypeStruct((E, H), jnp.float32),
    )(edge_feats, gsum, e2g_2d, fd, lat, wef, wlat, wfd, be1, we2, be2)


def _scatter_body(e2_hbm, src2d, zrows, zcnt, ones_hbm, agg_out, cnt_out,
                  agg_sh, cnt_sh, idx_v, rows_v, ones_v):
    c = lax.axis_index("c")
    s = lax.axis_index("s")
    wid = c * 16 + s
    pltpu.sync_copy(zrows.at[pl.ds(s * NPT, NPT), :], agg_sh.at[pl.ds(s * NPT, NPT), :])
    pltpu.sync_copy(zcnt.at[pl.ds(s * NPT, NPT), :], cnt_sh.at[pl.ds(s * NPT, NPT), :])
    pltpu.sync_copy(ones_hbm, ones_v)
    plsc.subcore_barrier()
    pltpu.sync_copy(src2d.at[pl.ds(wid * NCH, NCH), :], idx_v)

    def chunk(j, _):
        eoff = wid * EPW + j * CH
        pltpu.sync_copy(e2_hbm.at[pl.ds(eoff, CH), :], rows_v)
        pltpu.sync_copy(rows_v, agg_sh.at[idx_v.at[j]], add=True)
        pltpu.sync_copy(ones_v, cnt_sh.at[idx_v.at[j]], add=True)
        return ()

    lax.fori_loop(0, NCH, chunk, ())
    plsc.subcore_barrier()
    pltpu.sync_copy(agg_sh.at[pl.ds(s * NPT, NPT), :], agg_out.at[c, pl.ds(s * NPT, NPT), :])
    pltpu.sync_copy(cnt_sh.at[pl.ds(s * NPT, NPT), :], cnt_out.at[c, pl.ds(s * NPT, NPT), :])


def _scatter_mean_partials(e2, src2d, zrows, zcnt, ones16):
    mesh = plsc.VectorSubcoreMesh(core_axis_name="c", subcore_axis_name="s")
    fn = pl.kernel(
        _scatter_body,
        out_type=[
            jax.ShapeDtypeStruct((2, N, H), jnp.float32),
            jax.ShapeDtypeStruct((2, N, 16), jnp.float32),
        ],
        mesh=mesh,
        scratch_types=[
            pltpu.VMEM_SHARED((N, H), jnp.float32),
            pltpu.VMEM_SHARED((N, 16), jnp.float32),
            pltpu.VMEM((NCH, CH), jnp.int32),
            pltpu.VMEM((CH, H), jnp.float32),
            pltpu.VMEM((CH, 16), jnp.float32),
        ],
        compiler_params=pltpu.CompilerParams(use_tc_tiling_on_sc=False),
    )
    return fn(e2, src2d, zrows, zcnt, ones16)


def _node_mlp_body(nf, h, aggp, cntp, wn1a, wn1b, bn1, wn2, bn2, out):
    agg = aggp[0] + aggp[1]
    cnt = cntp[0, :, 0:1] + cntp[1, :, 0:1]
    agg = agg / jnp.maximum(cnt, 1.0)
    n1 = jax.nn.silu(
        jnp.dot(h[...], wn1a[...], preferred_element_type=jnp.float32)
        + jnp.dot(agg, wn1b[...], preferred_element_type=jnp.float32)
        + bn1[...])
    n2 = jax.nn.silu(jnp.dot(n1, wn2[...], preferred_element_type=jnp.float32)
                     + bn2[...])
    out[...] = nf[...] + n2


def _node_mlp(nf, h, aggp, cntp, wn1a, wn1b, bn1, wn2, bn2):
    const = pl.BlockSpec((H, H), lambda i: (0, 0))
    vec = pl.BlockSpec((1, H), lambda i: (0, 0))
    blk = pl.BlockSpec((BN, H), lambda i: (i, 0))
    return pl.pallas_call(
        _node_mlp_body,
        grid=(N // BN,),
        in_specs=[
            blk,
            blk,
            pl.BlockSpec((2, BN, H), lambda i: (0, i, 0)),
            pl.BlockSpec((2, BN, 16), lambda i: (0, i, 0)),
            const, const, vec, const, vec,
        ],
        out_specs=blk,
        out_shape=jax.ShapeDtypeStruct((N, H), jnp.float32),
    )(nf, h, aggp, cntp, wn1a, wn1b, bn1, wn2, bn2)


def kernel(node_features, lattices, edge_index, edge2graph, frac_diff,
           edge_feats, ln_gamma, ln_beta, W_e1, b_e1, W_e2, b_e2,
           W_n1, b_n1, W_n2, b_n2):
    f32 = jnp.float32
    W_ef = W_e1[0:H]
    W_hi = W_e1[H:2 * H]
    W_hj = W_e1[2 * H:3 * H]
    W_lat = W_e1[3 * H:3 * H + 6]
    W_fd = W_e1[3 * H + 6:]
    src = edge_index[0].astype(jnp.int32)
    dst = edge_index[1].astype(jnp.int32)
    src2d = src.reshape(E // CH, CH)
    dst2d = dst.reshape(E // CH, CH)
    e2g_2d = edge2graph.astype(jnp.int32).reshape(E, 1)
    g2 = ln_gamma.reshape(1, H).astype(f32)
    b2 = ln_beta.reshape(1, H).astype(f32)

    h, pi, pj = _node_pre(node_features, g2, b2, W_hi, W_hj)
    gsum = _gather_sum(pi, pj, src2d, dst2d)
    e2 = _edge_mlp(edge_feats, gsum, e2g_2d, frac_diff,
                   lattices.reshape(G, 6), W_ef, W_lat, W_fd,
                   b_e1.reshape(1, H), W_e2, b_e2.reshape(1, H))
    zrows = jnp.zeros((N, H), f32)
    zcnt = jnp.zeros((N, 16), f32)
    ones16 = jnp.ones((CH, 16), f32)
    aggp, cntp = _scatter_mean_partials(e2, src2d, zrows, zcnt, ones16)
    out = _node_mlp(node_features, h, aggp, cntp,
                    W_n1[0:H], W_n1[H:2 * H], b_n1.reshape(1, H),
                    W_n2, b_n2.reshape(1, H))
    return out


# speedup vs baseline: 1.4484x; 1.4484x over previous
import functools

import jax
import jax.numpy as jnp
from jax import lax
from jax.experimental import pallas as pl
from jax.experimental.pallas import tpu as pltpu
from jax.experimental.pallas import tpu_sc as plsc

N, E, G, H = 10000, 320000, 32, 128
NW = 32
CH = 125
NSPLIT = 4
EP = E // NSPLIT
RPP = EP // CH
CPW = RPP // NW
ND = 2
PPD = NSPLIT // ND
NPT = N // 16

BN = 1000
BE = 2000

_SC_PARAMS = pltpu.CompilerParams(use_tc_tiling_on_sc=False)


def _ln_block(x, g, b, eps=1e-5):
    m = jnp.mean(x, axis=-1, keepdims=True)
    v = jnp.mean((x - m) ** 2, axis=-1, keepdims=True)
    return (x - m) / jnp.sqrt(v + eps) * g + b


def _node_pre_body(nf, g2, b2, whi, whj, h_out, pi_out, pj_out):
    h = _ln_block(nf[...], g2[...], b2[...])
    h_out[...] = h
    pi_out[...] = jnp.dot(h, whi[...], preferred_element_type=jnp.float32)
    pj_out[...] = jnp.dot(h, whj[...], preferred_element_type=jnp.float32)


def _node_pre(nf, g2, b2, whi, whj):
    const = pl.BlockSpec((H, H), lambda i: (0, 0))
    vec = pl.BlockSpec((1, H), lambda i: (0, 0))
    blk = pl.BlockSpec((BN, H), lambda i: (i, 0))
    return pl.pallas_call(
        _node_pre_body,
        grid=(N // BN,),
        in_specs=[blk, vec, vec, const, const],
        out_specs=[blk, blk, blk],
        out_shape=[jax.ShapeDtypeStruct((N, H), jnp.float32)] * 3,
    )(nf, g2, b2, whi, whj)


def _gather_body(part, pi_hbm, pj_hbm, src2d, dst2d, out_hbm, idx_s, idx_d, rows):
    wid = lax.axis_index("c") * 16 + lax.axis_index("s")
    row0 = part * RPP + wid * CPW
    pltpu.sync_copy(src2d.at[pl.ds(row0, CPW), :], idx_s)
    pltpu.sync_copy(dst2d.at[pl.ds(row0, CPW), :], idx_d)

    def chunk(j, _):
        eoff = wid * (CPW * CH) + j * CH
        pltpu.sync_copy(pi_hbm.at[idx_s.at[j]], rows)
        pltpu.sync_copy(pj_hbm.at[idx_d.at[j]], rows, add=True)
        pltpu.sync_copy(rows, out_hbm.at[pl.ds(eoff, CH), :])
        return ()

    lax.fori_loop(0, CPW, chunk, ())


def _gather_sum(part, pi, pj, src2d, dst2d):
    mesh = plsc.VectorSubcoreMesh(core_axis_name="c", subcore_axis_name="s")
    fn = pl.kernel(
        functools.partial(_gather_body, part),
        out_type=jax.ShapeDtypeStruct((EP, H), jnp.float32),
        mesh=mesh,
        scratch_types=[
            pltpu.VMEM((CPW, CH), jnp.int32),
            pltpu.VMEM((CPW, CH), jnp.int32),
            pltpu.VMEM((CH, H), jnp.float32),
        ],
        compiler_params=_SC_PARAMS,
        name=f"gather_sum_p{part}",
    )
    return fn(pi, pj, src2d, dst2d)


def _edge_body(ef, gsum, e2g, fd, lat, wef, wlat, wfd, be1, we2, be2, out):
    qb = jnp.dot(lat[...], wlat[...], preferred_element_type=jnp.float32)
    onehot = (e2g[...] == lax.broadcasted_iota(jnp.int32, (1, G), 1)).astype(jnp.float32)
    lat_term = jnp.dot(onehot, qb, preferred_element_type=jnp.float32)
    fdv = fd[...]
    wfdv = wfd[...]
    fd_term = (fdv[:, 0:1] * wfdv[0:1, :] + fdv[:, 1:2] * wfdv[1:2, :]
               + fdv[:, 2:3] * wfdv[2:3, :])
    pre = (jnp.dot(ef[...], wef[...], preferred_element_type=jnp.float32)
           + gsum[...] + lat_term + fd_term + be1[...])
    e1 = jax.nn.silu(pre)
    e2 = jax.nn.silu(jnp.dot(e1, we2[...], preferred_element_type=jnp.float32)
                     + be2[...])
    out[...] = e2


def _edge_mlp(part, edge_feats, gsum, e2g_2d, fd, lat, wef, wlat, wfd, be1, we2, be2):
    off = part * (EP // BE)
    const = pl.BlockSpec((H, H), lambda i: (0, 0))
    vec = pl.BlockSpec((1, H), lambda i: (0, 0))
    return pl.pallas_call(
        _edge_body,
        grid=(EP // BE,),
        in_specs=[
            pl.BlockSpec((BE, H), lambda i: (off + i, 0)),
            pl.BlockSpec((BE, H), lambda i: (i, 0)),
            pl.BlockSpec((BE, 1), lambda i: (off + i, 0)),
            pl.BlockSpec((BE, 3), lambda i: (off + i, 0)),
            pl.BlockSpec((G, 6), lambda i: (0, 0)),
            const,
            pl.BlockSpec((6, H), lambda i: (0, 0)),
            pl.BlockSpec((3, H), lambda i: (0, 0)),
            vec,
            const,
            vec,
        ],
        out_specs=pl.BlockSpec((BE, H), lambda i: (i, 0)),
        out_shape=jax.ShapeDtypeStruct((EP, H), jnp.float32),
        name=f"edge_mlp_p{part}",
    )(edge_feats, gsum, e2g_2d, fd, lat, wef, wlat, wfd, be1, we2, be2)


def _scatter_body(dpart, e2a_hbm, e2b_hbm, src2d, zrows, zcnt, ones_hbm,
                  agg_out, cnt_out, agg_sh, cnt_sh, idx_v, rows_v, ones_v):
    c = lax.axis_index("c")
    s = lax.axis_index("s")
    wid = c * 16 + s
    pltpu.sync_copy(zrows.at[pl.ds(s * NPT, NPT), :], agg_sh.at[pl.ds(s * NPT, NPT), :])
    pltpu.sync_copy(zcnt.at[pl.ds(s * NPT, NPT), :], cnt_sh.at[pl.ds(s * NPT, NPT), :])
    pltpu.sync_copy(ones_hbm, ones_v)
    plsc.subcore_barrier()
    for pl_idx in range(PPD):
        part = dpart * PPD + pl_idx
        pltpu.sync_copy(src2d.at[pl.ds(part * RPP + wid * CPW, CPW), :],
                        idx_v.at[pl.ds(pl_idx * CPW, CPW), :])

    def chunk_of(e2_hbm, pl_idx):
        def chunk(j, _):
            eoff = wid * (CPW * CH) + j * CH
            pltpu.sync_copy(e2_hbm.at[pl.ds(eoff, CH), :], rows_v)
            pltpu.sync_copy(rows_v, agg_sh.at[idx_v.at[pl_idx * CPW + j]], add=True)
            pltpu.sync_copy(ones_v, cnt_sh.at[idx_v.at[pl_idx * CPW + j]], add=True)
            return ()
        return chunk

    lax.fori_loop(0, CPW, chunk_of(e2a_hbm, 0), ())
    lax.fori_loop(0, CPW, chunk_of(e2b_hbm, 1), ())
    plsc.subcore_barrier()
    pltpu.sync_copy(agg_sh.at[pl.ds(s * NPT, NPT), :], agg_out.at[c, pl.ds(s * NPT, NPT), :])
    pltpu.sync_copy(cnt_sh.at[pl.ds(s * NPT, NPT), :], cnt_out.at[c, pl.ds(s * NPT, NPT), :])


def _scatter_mean_partials(dpart, e2a, e2b, src2d, zrows, zcnt, ones16):
    mesh = plsc.VectorSubcoreMesh(core_axis_name="c", subcore_axis_name="s")
    fn = pl.kernel(
        functools.partial(_scatter_body, dpart),
        out_type=[
            jax.ShapeDtypeStruct((2, N, H), jnp.float32),
            jax.ShapeDtypeStruct((2, N, 16), jnp.float32),
        ],
        mesh=mesh,
        scratch_types=[
            pltpu.VMEM_SHARED((N, H), jnp.float32),
            pltpu.VMEM_SHARED((N, 16), jnp.float32),
            pltpu.VMEM((PPD * CPW, CH), jnp.int32),
            pltpu.VMEM((CH, H), jnp.float32),
            pltpu.VMEM((CH, 16), jnp.float32),
        ],
        compiler_params=_SC_PARAMS,
        name=f"scatter_add_d{dpart}",
    )
    return fn(e2a, e2b, src2d, zrows, zcnt, ones16)


def _node_mlp_body(nf, h, aggp0, aggp1, cntp0, cntp1, wn1a, wn1b, bn1, wn2, bn2, out):
    agg = aggp0[0] + aggp0[1] + aggp1[0] + aggp1[1]
    cnt = (cntp0[0, :, 0:1] + cntp0[1, :, 0:1]
           + cntp1[0, :, 0:1] + cntp1[1, :, 0:1])
    agg = agg / jnp.maximum(cnt, 1.0)
    n1 = jax.nn.silu(
        jnp.dot(h[...], wn1a[...], preferred_element_type=jnp.float32)
        + jnp.dot(agg, wn1b[...], preferred_element_type=jnp.float32)
        + bn1[...])
    n2 = jax.nn.silu(jnp.dot(n1, wn2[...], preferred_element_type=jnp.float32)
                     + bn2[...])
    out[...] = nf[...] + n2


def _node_mlp(nf, h, aggps, cntps, wn1a, wn1b, bn1, wn2, bn2):
    const = pl.BlockSpec((H, H), lambda i: (0, 0))
    vec = pl.BlockSpec((1, H), lambda i: (0, 0))
    blk = pl.BlockSpec((BN, H), lambda i: (i, 0))
    pblk = pl.BlockSpec((2, BN, H), lambda i: (0, i, 0))
    cblk = pl.BlockSpec((2, BN, 16), lambda i: (0, i, 0))
    return pl.pallas_call(
        _node_mlp_body,
        grid=(N // BN,),
        in_specs=[blk, blk, pblk, pblk, cblk, cblk, const, const, vec, const, vec],
        out_specs=blk,
        out_shape=jax.ShapeDtypeStruct((N, H), jnp.float32),
    )(nf, h, aggps[0], aggps[1], cntps[0], cntps[1], wn1a, wn1b, bn1, wn2, bn2)


def kernel(node_features, lattices, edge_index, edge2graph, frac_diff,
           edge_feats, ln_gamma, ln_beta, W_e1, b_e1, W_e2, b_e2,
           W_n1, b_n1, W_n2, b_n2):
    f32 = jnp.float32
    W_ef = W_e1[0:H]
    W_hi = W_e1[H:2 * H]
    W_hj = W_e1[2 * H:3 * H]
    W_lat = W_e1[3 * H:3 * H + 6]
    W_fd = W_e1[3 * H + 6:]
    src = edge_index[0].astype(jnp.int32)
    dst = edge_index[1].astype(jnp.int32)
    src2d = src.reshape(E // CH, CH)
    dst2d = dst.reshape(E // CH, CH)
    e2g_2d = edge2graph.astype(jnp.int32).reshape(E, 1)
    g2 = ln_gamma.reshape(1, H).astype(f32)
    b2 = ln_beta.reshape(1, H).astype(f32)
    lat6 = lattices.reshape(G, 6)
    be1 = b_e1.reshape(1, H)
    be2 = b_e2.reshape(1, H)

    h, pi, pj = _node_pre(node_features, g2, b2, W_hi, W_hj)
    e2_parts = []
    for part in range(NSPLIT):
        gsum = _gather_sum(part, pi, pj, src2d, dst2d)
        e2_parts.append(_edge_mlp(part, edge_feats, gsum, e2g_2d, frac_diff,
                                  lat6, W_ef, W_lat, W_fd, be1, W_e2, be2))
    zrows = jnp.zeros((N, H), f32)
    zcnt = jnp.zeros((N, 16), f32)
    ones16 = jnp.ones((CH, 16), f32)
    aggps, cntps = [], []
    for dpart in range(ND):
        aggp, cntp = _scatter_mean_partials(
            dpart, e2_parts[dpart * PPD], e2_parts[dpart * PPD + 1],
            src2d, zrows, zcnt, ones16)
        aggps.append(aggp)
        cntps.append(cntp)
    out = _node_mlp(node_features, h, aggps, cntps,
                    W_n1[0:H], W_n1[H:2 * H], b_n1.reshape(1, H),
                    W_n2, b_n2.reshape(1, H))
    return out
